# baseline (speedup 1.0000x reference)
import functools
import math

import jax
import jax.numpy as jnp
import numpy as np
from jax import lax
from jax.experimental import pallas as pl
from jax.experimental.pallas import tpu as pltpu
from jax.experimental.pallas import tpu_sc as plsc

_PATCH = 16
_EMBED = 768
_HW = 512
_GRID = _HW // _PATCH
_N = _GRID * _GRID
_WINDOW = 7
_NWIN = 4
_MASK_RATIO = 0.8
_RT = 256
_NG = _N // _RT
_NBUF = 3

_NWORK = 32
_KG = 48
_SUB = 128
_JPP = 24
_PH = 4


def _mask_array():
    H = W_ = _GRID
    all_inds = jnp.arange(H * W_, dtype=jnp.int32).reshape(H, W_)
    pad = _WINDOW // 2
    selectable = all_inds[pad:-pad, pad:-pad].reshape(-1)
    key = jax.random.key(42)
    sampled = jax.random.choice(key, selectable.shape[0], (_NWIN,), replace=False)
    centroids = selectable[sampled]
    off = jnp.arange(int(math.ceil(-_WINDOW / 2)), int(math.ceil(_WINDOW / 2)),
                     dtype=jnp.int32)
    wo = jnp.tile(off[None, :], (_WINDOW, 1))
    sq = jnp.tile((off * H)[None, :], (_WINDOW, 1)).T
    wo = (wo + sq).reshape(1, -1)
    coords = jnp.tile(centroids[:, None], (1, _WINDOW ** 2)) + wo
    n_mask = int(_MASK_RATIO * _WINDOW ** 2)
    inds = coords[:, :n_mask]
    mask = jnp.zeros((_NWIN, _N), jnp.float32)
    mask = mask.at[jnp.arange(_NWIN)[:, None], inds].set(1.0)
    return mask.T.reshape(_NG, _RT, _NWIN)


with jax.default_device(jax.devices("cpu")[0]):
    _MASK_CONST = np.asarray(_mask_array())


_QP = 4
_PPH = 32
_SPH = 8
_DP = _SPH // 2


def _in_copy(x_hbm, in_v, isem, b, r, slot, c):
    return pltpu.make_async_copy(
        x_hbm.at[pl.ds((b * 3 + c) * _HW + r * _PATCH, _PATCH), :],
        in_v.at[slot, pl.ds(c * _PATCH, _PATCH), :],
        isem.at[slot])


def _out_copy(out_hbm, out_v, osem, b, n0, slot):
    return pltpu.make_async_copy(
        out_v.at[slot], out_hbm.at[b, pl.ds(n0, _PPH), :], osem.at[slot])


def _shuffle_body(x_hbm, out_hbm, in_v, out_v, isem, osem):
    wid = lax.axis_index("s") * 2 + lax.axis_index("c")
    b = wid // _QP
    q = wid % _QP
    r_base = q * _SPH
    n_base = q * (_SPH * _PPH)

    for slot in range(2):
        for c in range(3):
            _in_copy(x_hbm, in_v, isem, b, r_base + slot, slot, c).start()

    def _double_phase(dp, carry):
        for slot in range(2):
            ph = dp * 2 + slot
            r = r_base + ph
            n0 = n_base + ph * _PPH
            for c in range(3):
                _in_copy(x_hbm, in_v, isem, b, r, slot, c).wait()

            @pl.when(dp >= 1)
            def _():
                _out_copy(out_hbm, out_v, osem, b, n0 - 2 * _PPH, slot).wait()

            for s in range(_PPH):
                for c in range(3):
                    for i2 in range(_PATCH):
                        vec = in_v[slot, c * _PATCH + i2,
                                   pl.ds(s * _PATCH, _PATCH)]
                        out_v[slot, s,
                              pl.ds((c * _PATCH + i2) * _PATCH, _PATCH)] = vec

            _out_copy(out_hbm, out_v, osem, b, n0, slot).start()

            @pl.when(dp < _DP - 1)
            def _():
                for c in range(3):
                    _in_copy(x_hbm, in_v, isem, b, r + 2, slot, c).start()
        return carry

    lax.fori_loop(0, _DP, _double_phase, 0)
    for slot in range(2):
        ph = _SPH - 2 + slot
        _out_copy(out_hbm, out_v, osem, b, n_base + ph * _PPH, slot).wait()


def _im2col_sc(x, Bn):
    x2d = x.reshape(Bn * 3 * _HW, _HW)
    mesh = plsc.VectorSubcoreMesh(core_axis_name="c", subcore_axis_name="s")
    xp = pl.kernel(
        _shuffle_body,
        out_type=jax.ShapeDtypeStruct((Bn, _N, _KG * _PATCH), jnp.float32),
        mesh=mesh,
        scratch_types=[
            pltpu.VMEM((2, 3 * _PATCH, _HW), jnp.float32),
            pltpu.VMEM((2, _PPH, _KG * _PATCH), jnp.float32),
            pltpu.SemaphoreType.DMA((2,)),
            pltpu.SemaphoreType.DMA((2,)),
        ],
    )(x2d)
    return xp


def _body(xp_ref, w_ref, b_ref, tok_ref, mask_ref, out_hbm, bufs, sems):
    b = pl.program_id(0)
    i = pl.program_id(1)
    step = b * _NG + i
    nsteps = pl.num_programs(0) * pl.num_programs(1)
    slot = step % _NBUF

    @pl.when(step >= _NBUF)
    def _():
        for w in range(_NWIN):
            pltpu.make_async_copy(
                bufs.at[slot, w],
                out_hbm.at[b, w, pl.ds(i * _RT, _RT), :],
                sems.at[slot, w]).wait()

    acc = jnp.dot(xp_ref[0], w_ref[...], preferred_element_type=jnp.float32)
    acc = acc + b_ref[...]
    tok = jnp.broadcast_to(tok_ref[...], acc.shape)
    m = mask_ref[0]
    for w in range(_NWIN):
        sel = jnp.broadcast_to(m[:, w:w + 1] != 0.0, acc.shape)
        bufs[slot, w] = jnp.where(sel, tok, acc)
        pltpu.make_async_copy(
            bufs.at[slot, w],
            out_hbm.at[b, w, pl.ds(i * _RT, _RT), :],
            sems.at[slot, w]).start()

    @pl.when(step == nsteps - 1)
    def _():
        for s in range(_NBUF):
            for w in range(_NWIN):
                pltpu.make_async_copy(
                    bufs.at[s, w],
                    out_hbm.at[b, w, pl.ds(i * _RT, _RT), :],
                    sems.at[s, w]).wait()


def kernel(x, W, b, mask_token):
    Bn = x.shape[0]
    p = _PATCH
    xp = _im2col_sc(x, Bn)
    Wm = W.reshape(_EMBED, 3 * p * p).T
    mask = jnp.asarray(_MASK_CONST)
    tok = mask_token.reshape(1, _EMBED)
    b2 = b.reshape(1, _EMBED)

    out = pl.pallas_call(
        _body,
        grid=(Bn, _NG),
        in_specs=[
            pl.BlockSpec((1, _RT, 3 * p * p), lambda bi, i: (bi, i, 0)),
            pl.BlockSpec((3 * p * p, _EMBED), lambda bi, i: (0, 0)),
            pl.BlockSpec((1, _EMBED), lambda bi, i: (0, 0)),
            pl.BlockSpec((1, _EMBED), lambda bi, i: (0, 0)),
            pl.BlockSpec((1, _RT, _NWIN), lambda bi, i: (i, 0, 0)),
        ],
        out_specs=pl.BlockSpec(memory_space=pl.ANY),
        out_shape=jax.ShapeDtypeStruct((Bn, _NWIN, _N, _EMBED), jnp.float32),
        scratch_shapes=[
            pltpu.VMEM((_NBUF, _NWIN, _RT, _EMBED), jnp.float32),
            pltpu.SemaphoreType.DMA((_NBUF, _NWIN)),
        ],
        compiler_params=pltpu.CompilerParams(
            dimension_semantics=("arbitrary", "arbitrary")),
    )(xp, Wm, b2, tok, mask)
    return out


# speedup vs baseline: 1.0354x; 1.0354x over previous
import functools
import math

import jax
import jax.numpy as jnp
import numpy as np
from jax import lax
from jax.experimental import pallas as pl
from jax.experimental.pallas import tpu as pltpu
from jax.experimental.pallas import tpu_sc as plsc

_PATCH = 16
_EMBED = 768
_HW = 512
_GRID = _HW // _PATCH
_N = _GRID * _GRID
_WINDOW = 7
_NWIN = 4
_MASK_RATIO = 0.8
_RT = 256
_NG = _N // _RT
_NBUF = 3

_NWORK = 32
_KG = 48
_SUB = 128
_JPP = 24
_PH = 4


def _mask_array():
    H = W_ = _GRID
    all_inds = jnp.arange(H * W_, dtype=jnp.int32).reshape(H, W_)
    pad = _WINDOW // 2
    selectable = all_inds[pad:-pad, pad:-pad].reshape(-1)
    key = jax.random.key(42)
    sampled = jax.random.choice(key, selectable.shape[0], (_NWIN,), replace=False)
    centroids = selectable[sampled]
    off = jnp.arange(int(math.ceil(-_WINDOW / 2)), int(math.ceil(_WINDOW / 2)),
                     dtype=jnp.int32)
    wo = jnp.tile(off[None, :], (_WINDOW, 1))
    sq = jnp.tile((off * H)[None, :], (_WINDOW, 1)).T
    wo = (wo + sq).reshape(1, -1)
    coords = jnp.tile(centroids[:, None], (1, _WINDOW ** 2)) + wo
    n_mask = int(_MASK_RATIO * _WINDOW ** 2)
    inds = coords[:, :n_mask]
    mask = jnp.zeros((_NWIN, _N), jnp.float32)
    mask = mask.at[jnp.arange(_NWIN)[:, None], inds].set(1.0)
    return mask.T.reshape(_NG, _RT, _NWIN)


with jax.default_device(jax.devices("cpu")[0]):
    _MASK_CONST = np.asarray(_mask_array())


_PPH = 32


def _in_copy(x_hbm, in_v, isem, b, r, slot, c):
    return pltpu.make_async_copy(
        x_hbm.at[pl.ds((b * 3 + c) * _HW + r * _PATCH, _PATCH), :],
        in_v.at[slot, pl.ds(c * _PATCH, _PATCH), :],
        isem.at[slot])


def _out_copy(out_hbm, out_v, osem, b, n0, slot):
    return pltpu.make_async_copy(
        out_v.at[slot], out_hbm.at[b, pl.ds(n0, _PPH), :], osem.at[slot])


def _make_shuffle_body(Bh, b_off):
    qp = _NWORK // Bh
    sph = _GRID // qp
    dp_trips = sph // 2

    def _shuffle_body(x_hbm, out_hbm, in_v, out_v, isem, osem):
        wid = lax.axis_index("s") * 2 + lax.axis_index("c")
        b_local = wid // qp
        b = b_local + b_off
        q = wid % qp
        r_base = q * sph
        n_base = q * (sph * _PPH)

        for slot in range(2):
            for c in range(3):
                _in_copy(x_hbm, in_v, isem, b, r_base + slot, slot, c).start()

        def _double_phase(dp, carry):
            for slot in range(2):
                ph = dp * 2 + slot
                r = r_base + ph
                n0 = n_base + ph * _PPH
                for c in range(3):
                    _in_copy(x_hbm, in_v, isem, b, r, slot, c).wait()

                @pl.when(dp >= 1)
                def _():
                    _out_copy(out_hbm, out_v, osem, b_local,
                              n0 - 2 * _PPH, slot).wait()

                for s in range(_PPH):
                    for c in range(3):
                        for i2 in range(_PATCH):
                            vec = in_v[slot, c * _PATCH + i2,
                                       pl.ds(s * _PATCH, _PATCH)]
                            out_v[slot, s,
                                  pl.ds((c * _PATCH + i2) * _PATCH,
                                        _PATCH)] = vec

                _out_copy(out_hbm, out_v, osem, b_local, n0, slot).start()

                @pl.when(dp < dp_trips - 1)
                def _():
                    for c in range(3):
                        _in_copy(x_hbm, in_v, isem, b, r + 2, slot, c).start()
            return carry

        lax.fori_loop(0, dp_trips, _double_phase, 0)
        for slot in range(2):
            ph = sph - 2 + slot
            _out_copy(out_hbm, out_v, osem, b_local,
                      n_base + ph * _PPH, slot).wait()

    return _shuffle_body


def _im2col_sc(x2d, Bh, b_off):
    mesh = plsc.VectorSubcoreMesh(core_axis_name="c", subcore_axis_name="s")
    xp = pl.kernel(
        _make_shuffle_body(Bh, b_off),
        out_type=jax.ShapeDtypeStruct((Bh, _N, _KG * _PATCH), jnp.float32),
        mesh=mesh,
        scratch_types=[
            pltpu.VMEM((2, 3 * _PATCH, _HW), jnp.float32),
            pltpu.VMEM((2, _PPH, _KG * _PATCH), jnp.float32),
            pltpu.SemaphoreType.DMA((2,)),
            pltpu.SemaphoreType.DMA((2,)),
        ],
    )(x2d)
    return xp


def _make_tc_body(b_off, Bh, has_obuf):
    def _body(*refs):
        if has_obuf:
            xp_ref, w_ref, b_ref, tok_ref, mask_ref, _obuf, out_hbm, \
                bufs, sems = refs
        else:
            xp_ref, w_ref, b_ref, tok_ref, mask_ref, out_hbm, bufs, sems = refs
        b = pl.program_id(0)
        i = pl.program_id(1)
        bo = b + b_off
        step = b * _NG + i
        nsteps = Bh * _NG
        slot = step % _NBUF

        @pl.when(step >= _NBUF)
        def _():
            for w in range(_NWIN):
                pltpu.make_async_copy(
                    bufs.at[slot, w],
                    out_hbm.at[bo, w, pl.ds(i * _RT, _RT), :],
                    sems.at[slot, w]).wait()

        acc = jnp.dot(xp_ref[0], w_ref[...],
                      preferred_element_type=jnp.float32)
        acc = acc + b_ref[...]
        tok = jnp.broadcast_to(tok_ref[...], acc.shape)
        m = mask_ref[0]
        for w in range(_NWIN):
            sel = jnp.broadcast_to(m[:, w:w + 1] != 0.0, acc.shape)
            bufs[slot, w] = jnp.where(sel, tok, acc)
            pltpu.make_async_copy(
                bufs.at[slot, w],
                out_hbm.at[bo, w, pl.ds(i * _RT, _RT), :],
                sems.at[slot, w]).start()

        @pl.when(step == nsteps - 1)
        def _():
            for s in range(_NBUF):
                for w in range(_NWIN):
                    pltpu.make_async_copy(
                        bufs.at[s, w],
                        out_hbm.at[bo, w, pl.ds(i * _RT, _RT), :],
                        sems.at[s, w]).wait()
    return _body


def _tc_half(xp, Wm, b2, tok, mask, obuf, b_off, Bn):
    Bh = xp.shape[0]
    in_specs = [
        pl.BlockSpec((1, _RT, _EMBED), lambda bi, i: (bi, i, 0)),
        pl.BlockSpec((_EMBED, _EMBED), lambda bi, i: (0, 0)),
        pl.BlockSpec((1, _EMBED), lambda bi, i: (0, 0)),
        pl.BlockSpec((1, _EMBED), lambda bi, i: (0, 0)),
        pl.BlockSpec((1, _RT, _NWIN), lambda bi, i: (i, 0, 0)),
    ]
    args = (xp, Wm, b2, tok, mask)
    aliases = {}
    if obuf is not None:
        in_specs.append(pl.BlockSpec(memory_space=pl.ANY))
        args += (obuf,)
        aliases = {5: 0}
    return pl.pallas_call(
        _make_tc_body(b_off, Bh, obuf is not None),
        grid=(Bh, _NG),
        in_specs=in_specs,
        out_specs=pl.BlockSpec(memory_space=pl.ANY),
        out_shape=jax.ShapeDtypeStruct((Bn, _NWIN, _N, _EMBED), jnp.float32),
        scratch_shapes=[
            pltpu.VMEM((_NBUF, _NWIN, _RT, _EMBED), jnp.float32),
            pltpu.SemaphoreType.DMA((_NBUF, _NWIN)),
        ],
        input_output_aliases=aliases,
        compiler_params=pltpu.CompilerParams(
            dimension_semantics=("arbitrary", "arbitrary")),
    )(*args)


def kernel(x, W, b, mask_token):
    Bn = x.shape[0]
    p = _PATCH
    Bh = Bn // 2
    Wm = W.reshape(_EMBED, 3 * p * p).T
    mask = jnp.asarray(_MASK_CONST)
    tok = mask_token.reshape(1, _EMBED)
    b2 = b.reshape(1, _EMBED)
    x2d = x.reshape(Bn * 3 * _HW, _HW)

    xp0 = _im2col_sc(x2d, Bh, 0)
    xp1 = _im2col_sc(x2d, Bh, Bh)
    out = _tc_half(xp0, Wm, b2, tok, mask, None, 0, Bn)
    out = _tc_half(xp1, Wm, b2, tok, mask, out, Bh, Bn)
    return out


# speedup vs baseline: 1.0389x; 1.0034x over previous
import math

import jax
import jax.numpy as jnp
import numpy as np
from jax import lax
from jax.experimental import pallas as pl
from jax.experimental.pallas import tpu as pltpu
from jax.experimental.pallas import tpu_sc as plsc

_PATCH = 16
_EMBED = 768
_HW = 512
_GRID = _HW // _PATCH
_N = _GRID * _GRID
_WINDOW = 7
_NWIN = 4
_MASK_RATIO = 0.8
_RT = 256
_NG = _N // _RT
_NBUF = 3

_NWORK = 32
_KG = 48


def _mask_array():
    H = W_ = _GRID
    all_inds = jnp.arange(H * W_, dtype=jnp.int32).reshape(H, W_)
    pad = _WINDOW // 2
    selectable = all_inds[pad:-pad, pad:-pad].reshape(-1)
    key = jax.random.key(42)
    sampled = jax.random.choice(key, selectable.shape[0], (_NWIN,), replace=False)
    centroids = selectable[sampled]
    off = jnp.arange(int(math.ceil(-_WINDOW / 2)), int(math.ceil(_WINDOW / 2)),
                     dtype=jnp.int32)
    wo = jnp.tile(off[None, :], (_WINDOW, 1))
    sq = jnp.tile((off * H)[None, :], (_WINDOW, 1)).T
    wo = (wo + sq).reshape(1, -1)
    coords = jnp.tile(centroids[:, None], (1, _WINDOW ** 2)) + wo
    n_mask = int(_MASK_RATIO * _WINDOW ** 2)
    inds = coords[:, :n_mask]
    mask = jnp.zeros((_NWIN, _N), jnp.float32)
    mask = mask.at[jnp.arange(_NWIN)[:, None], inds].set(1.0)
    return mask.T.reshape(_NG, _RT, _NWIN)


with jax.default_device(jax.devices("cpu")[0]):
    _MASK_CONST = np.asarray(_mask_array())


_PPH = 32


def _in_copy(x_hbm, in_v, isem, b, r, slot, c):
    return pltpu.make_async_copy(
        x_hbm.at[pl.ds((b * 3 + c) * _HW + r * _PATCH, _PATCH), :],
        in_v.at[slot, pl.ds(c * _PATCH, _PATCH), :],
        isem.at[slot])


def _out_copy(out_hbm, out_v, osem, b, n0, slot):
    return pltpu.make_async_copy(
        out_v.at[slot], out_hbm.at[b, pl.ds(n0, _PPH), :], osem.at[slot])


def _make_shuffle_body(Bh, b_off):
    qp = _NWORK // Bh
    sph = _GRID // qp
    dp_trips = sph // 2

    def _shuffle_body(x_hbm, out_hbm, in_v, out_v, isem, osem):
        wid = lax.axis_index("s") * 2 + lax.axis_index("c")
        b_local = wid // qp
        b = b_local + b_off
        q = wid % qp
        r_base = q * sph
        n_base = q * (sph * _PPH)

        for slot in range(2):
            for c in range(3):
                _in_copy(x_hbm, in_v, isem, b, r_base + slot, slot, c).start()

        def _double_phase(dp, carry):
            for slot in range(2):
                ph = dp * 2 + slot
                r = r_base + ph
                n0 = n_base + ph * _PPH
                for c in range(3):
                    _in_copy(x_hbm, in_v, isem, b, r, slot, c).wait()

                @pl.when(dp >= 1)
                def _():
                    _out_copy(out_hbm, out_v, osem, b_local,
                              n0 - 2 * _PPH, slot).wait()

                for s in range(_PPH):
                    for c in range(3):
                        for i2 in range(_PATCH):
                            vec = in_v[slot, c * _PATCH + i2,
                                       pl.ds(s * _PATCH, _PATCH)]
                            out_v[slot, s,
                                  pl.ds((c * _PATCH + i2) * _PATCH,
                                        _PATCH)] = vec

                _out_copy(out_hbm, out_v, osem, b_local, n0, slot).start()

                @pl.when(dp < dp_trips - 1)
                def _():
                    for c in range(3):
                        _in_copy(x_hbm, in_v, isem, b, r + 2, slot, c).start()
            return carry

        lax.fori_loop(0, dp_trips, _double_phase, 0)
        for slot in range(2):
            ph = sph - 2 + slot
            _out_copy(out_hbm, out_v, osem, b_local,
                      n_base + ph * _PPH, slot).wait()

    return _shuffle_body


def _im2col_sc(x2d, Bh, b_off):
    mesh = plsc.VectorSubcoreMesh(core_axis_name="c", subcore_axis_name="s")
    xp = pl.kernel(
        _make_shuffle_body(Bh, b_off),
        out_type=jax.ShapeDtypeStruct((Bh, _N, _KG * _PATCH), jnp.float32),
        mesh=mesh,
        scratch_types=[
            pltpu.VMEM((2, 3 * _PATCH, _HW), jnp.float32),
            pltpu.VMEM((2, _PPH, _KG * _PATCH), jnp.float32),
            pltpu.SemaphoreType.DMA((2,)),
            pltpu.SemaphoreType.DMA((2,)),
        ],
    )(x2d)
    return xp


def _make_tc_body(b_off, Bh, has_obuf):
    def _body(*refs):
        if has_obuf:
            xp_ref, w_ref, b_ref, tok_ref, mask_ref, _obuf, out_hbm, \
                bufs, sems = refs
        else:
            xp_ref, w_ref, b_ref, tok_ref, mask_ref, out_hbm, bufs, sems = refs
        b = pl.program_id(0)
        i = pl.program_id(1)
        bo = b + b_off
        step = b * _NG + i
        nsteps = Bh * _NG
        slot = step % _NBUF

        @pl.when(step >= _NBUF)
        def _():
            for w in range(_NWIN):
                pltpu.make_async_copy(
                    bufs.at[slot, w],
                    out_hbm.at[bo, w, pl.ds(i * _RT, _RT), :],
                    sems.at[slot, w]).wait()

        acc = jnp.dot(xp_ref[0], w_ref[...],
                      preferred_element_type=jnp.float32)
        acc = acc + b_ref[...]
        tok = jnp.broadcast_to(tok_ref[...], acc.shape)
        m = mask_ref[0]
        for w in range(_NWIN):
            sel = jnp.broadcast_to(m[:, w:w + 1] != 0.0, acc.shape)
            bufs[slot, w] = jnp.where(sel, tok, acc)
            pltpu.make_async_copy(
                bufs.at[slot, w],
                out_hbm.at[bo, w, pl.ds(i * _RT, _RT), :],
                sems.at[slot, w]).start()

        @pl.when(step == nsteps - 1)
        def _():
            for s in range(_NBUF):
                for w in range(_NWIN):
                    pltpu.make_async_copy(
                        bufs.at[s, w],
                        out_hbm.at[bo, w, pl.ds(i * _RT, _RT), :],
                        sems.at[s, w]).wait()
    return _body


def _tc_half(xp, Wm, b2, tok, mask, obuf, b_off, Bn):
    Bh = xp.shape[0]
    in_specs = [
        pl.BlockSpec((1, _RT, _EMBED), lambda bi, i: (bi, i, 0)),
        pl.BlockSpec((_EMBED, _EMBED), lambda bi, i: (0, 0)),
        pl.BlockSpec((1, _EMBED), lambda bi, i: (0, 0)),
        pl.BlockSpec((1, _EMBED), lambda bi, i: (0, 0)),
        pl.BlockSpec((1, _RT, _NWIN), lambda bi, i: (i, 0, 0)),
    ]
    args = (xp, Wm, b2, tok, mask)
    aliases = {}
    if obuf is not None:
        in_specs.append(pl.BlockSpec(memory_space=pl.ANY))
        args += (obuf,)
        aliases = {5: 0}
    return pl.pallas_call(
        _make_tc_body(b_off, Bh, obuf is not None),
        grid=(Bh, _NG),
        in_specs=in_specs,
        out_specs=pl.BlockSpec(memory_space=pl.ANY),
        out_shape=jax.ShapeDtypeStruct((Bn, _NWIN, _N, _EMBED), jnp.float32),
        scratch_shapes=[
            pltpu.VMEM((_NBUF, _NWIN, _RT, _EMBED), jnp.float32),
            pltpu.SemaphoreType.DMA((_NBUF, _NWIN)),
        ],
        input_output_aliases=aliases,
        compiler_params=pltpu.CompilerParams(
            dimension_semantics=("arbitrary", "arbitrary")),
    )(*args)


def kernel(x, W, b, mask_token):
    Bn = x.shape[0]
    p = _PATCH
    Bh = Bn // 2
    Wm = W.reshape(_EMBED, 3 * p * p).T
    mask = jnp.asarray(_MASK_CONST)
    tok = mask_token.reshape(1, _EMBED)
    b2 = b.reshape(1, _EMBED)
    x2d = x.reshape(Bn * 3 * _HW, _HW)

    xp0 = _im2col_sc(x2d, Bh, 0)
    xp1 = _im2col_sc(x2d, Bh, Bh)
    out = _tc_half(xp0, Wm, b2, tok, mask, None, 0, Bn)
    out = _tc_half(xp1, Wm, b2, tok, mask, out, Bh, Bn)
    return out
